# baseline (speedup 1.0000x reference)
import functools

import jax
import jax.numpy as jnp
from jax import lax
from jax.experimental import pallas as pl
from jax.experimental.pallas import tpu as pltpu
from jax.experimental.pallas import tpu_sc as plsc

N = 10000
E = 320000
IN_CH = 128
D1 = 64
D2 = 8
NG = 16
NCORES = 2
NSUB = 16
NTILES = NCORES * NSUB
EW = E // NTILES
G = 80
NGRP = EW // G
ZCH = 1000
NZCH = N // ZCH
NB = 5

_f32 = jnp.float32
_i32 = jnp.int32


def _tc_head_body(x_ref, w_ref, avs_ref, avd_ref, h_ref, asad_ref, cm_ref):
    h = jnp.dot(x_ref[...], w_ref[...], preferred_element_type=_f32)
    h_ref[...] = h
    a_s = jnp.sum(h * avs_ref[...], axis=1)
    a_d = jnp.sum(h * avd_ref[...], axis=1)
    asad_ref[0:1, :] = a_s[None]
    asad_ref[1:2, :] = a_d[None]
    cm = jnp.max(a_s) + jnp.max(a_d)
    cm_ref[...] = jnp.zeros((1, 16), _f32) + jnp.maximum(cm, 0.2 * cm)


def _tc_head(x, W1, a1s, a1d):
    return pl.pallas_call(
        _tc_head_body,
        out_shape=[
            jax.ShapeDtypeStruct((N, D1), _f32),
            jax.ShapeDtypeStruct((2, N), _f32),
            jax.ShapeDtypeStruct((1, 16), _f32),
        ],
    )(x, W1, a1s, a1d)


def _tc_mid_body(p_ref, b1_ref, w2_ref, avs_ref, avd_ref,
                 h2_ref, asad_ref, cm_ref):
    o = jnp.maximum(p_ref[0] + p_ref[1] + b1_ref[...], 0.0)
    h2 = jnp.dot(o, w2_ref[...], preferred_element_type=_f32)
    h2_ref[...] = h2
    a_s = jnp.sum(h2 * avs_ref[...], axis=1)
    a_d = jnp.sum(h2 * avd_ref[...], axis=1)
    asad_ref[0:1, :] = a_s[None]
    asad_ref[1:2, :] = a_d[None]
    cm = jnp.max(a_s) + jnp.max(a_d)
    cm_ref[...] = jnp.zeros((1, 16), _f32) + jnp.maximum(cm, 0.2 * cm)


def _tc_mid(p, b1, W2, a2s, a2d):
    return pl.pallas_call(
        _tc_mid_body,
        out_shape=[
            jax.ShapeDtypeStruct((N, D2), _f32),
            jax.ShapeDtypeStruct((2, N), _f32),
            jax.ShapeDtypeStruct((1, 16), _f32),
        ],
    )(p, b1, W2, a2s, a2d)


def _tc_end_body(p_ref, b2_ref, batch_ref, fcw_ref, fcb_ref, out_ref):
    h = jnp.maximum(p_ref[0] + p_ref[1] + b2_ref[...], 0.0)
    gids = lax.broadcasted_iota(_i32, (NG, N), 0)
    oh = (gids == batch_ref[...]).astype(_f32)
    counts = jnp.sum(oh, axis=1, keepdims=True)
    sums = jnp.dot(oh, h, preferred_element_type=_f32)
    pooled = sums / jnp.maximum(counts, 1.0)
    logits = jnp.sum(pooled * fcw_ref[...], axis=1, keepdims=True) + fcb_ref[...]
    out_ref[...] = 2.0 / (1.0 + jnp.exp(-logits)) - 1.0


def _tc_end(p, b2, batch2, fcw, fcb):
    return pl.pallas_call(
        _tc_end_body,
        out_shape=jax.ShapeDtypeStruct((NG, 1), _f32),
    )(p, b2, batch2, fcw, fcb)


_MESH = plsc.VectorSubcoreMesh(core_axis_name="c", subcore_axis_name="s")


def _sc_edge_pass(D, rows_scale_rowmode):
    def body(ei4, asad, cvec, feat, idc, zrows, out_parts,
             asv, adv, srcg, dstg, exv, denv,
             rows0, rows1, rows2, rows3, rows4, cvecv,
             den_sh, out_sh, sem, sems):
        rows = (rows0, rows1, rows2, rows3, rows4)
        c = lax.axis_index("c")
        s = lax.axis_index("s")
        w = c * NSUB + s

        pltpu.sync_copy(asad.at[0], asv)
        pltpu.sync_copy(asad.at[1], adv)
        pltpu.sync_copy(cvec.at[0], cvecv)
        cv = cvecv[...]

        zero16 = jnp.zeros((16,), _f32)

        def zero_den(i, carry):
            denv[pl.ds(i * 16, 16)] = zero16
            return carry
        lax.fori_loop(0, N // 16, zero_den, 0)

        @pl.when(s == 0)
        def _():
            pltpu.sync_copy(denv, den_sh)

        @pl.when(s < NZCH)
        def _():
            pltpu.sync_copy(zrows, out_sh.at[pl.ds(s * ZCH, ZCH)])
        plsc.subcore_barrier()

        for slab in (s + NSUB * (1 - c), s + NSUB * c):
          with jax.named_scope("p1_slab"):
            pltpu.sync_copy(ei4.at[0, slab], srcg)
            pltpu.sync_copy(ei4.at[1, slab], dstg)

            def p1(g, carry):
                for k in range(G // 16):
                    si = srcg[g, pl.ds(k * 16, 16)]
                    di = dstg[g, pl.ds(k * 16, 16)]
                    av = plsc.load_gather(asv, [si])
                    bv = plsc.load_gather(adv, [di])
                    e = av + bv
                    e = jnp.maximum(e, 0.2 * e)
                    ex = jnp.exp(e - cv)
                    exv[pl.ds(g * G + k * 16, 16)] = ex
                    plsc.addupdate_scatter(denv, [di], ex)
                return carry
            lax.fori_loop(0, NGRP, p1, 0)

        pltpu.sync_copy(idc, srcg)

        def dadd(i, carry):
            for b in range(NB):
                pltpu.async_copy(denv.at[pl.ds((i * NB + b) * G, G)],
                                 den_sh.at[srcg.at[i * NB + b]], sems,
                                 add=True)
            for b in range(NB):
                pltpu.make_async_copy(denv.at[pl.ds((i * NB + b) * G, G)],
                                      den_sh.at[srcg.at[i * NB + b]],
                                      sems).wait()
            return carry
        with jax.named_scope("dmerge"):
            lax.fori_loop(0, NGRP // NB, dadd, 0)
        plsc.subcore_barrier()

        pltpu.sync_copy(ei4.at[0, w], srcg)
        pltpu.sync_copy(ei4.at[1, w], dstg)
        pltpu.sync_copy(den_sh, denv)

        def aloop(g, carry):
            for k in range(G // 16):
                di = dstg[g, pl.ds(k * 16, 16)]
                d = plsc.load_gather(denv, [di])
                off = g * G + k * 16
                exv[pl.ds(off, 16)] = exv[pl.ds(off, 16)] / (d + 1e-16)
            return carry
        with jax.named_scope("alpha"):
            lax.fori_loop(0, NGRP, aloop, 0)

        if rows_scale_rowmode:
            def scale(rbuf, g):
                for k in range(G // 16):
                    av = exv[pl.ds(g * G + k * 16, 16)]
                    for e16 in range(16):
                        a = av[e16]
                        e = k * 16 + e16
                        for j in range(D // 16):
                            rbuf[e, pl.ds(j * 16, 16)] = rbuf[e, pl.ds(j * 16, 16)] * a
        else:
            def scale(rbuf, g):
                for k in range(G // 16):
                    av = exv[pl.ds(g * G + k * 16, 16)]
                    ridx = k * 16 + lax.iota(_i32, 16)
                    for j in range(D):
                        cj = jnp.full((16,), j, _i32)
                        col = plsc.load_gather(rbuf, [ridx, cj])
                        plsc.store_scatter(rbuf, [ridx, cj], col * av)

        for b in range(NB):
            pltpu.async_copy(feat.at[srcg.at[b]], rows[b], sem)

        def mloop(i, carry):
            g0 = i * NB
            for b in range(NB):
                g = g0 + b
                pltpu.make_async_copy(feat.at[srcg.at[g]], rows[b], sem).wait()
                scale(rows[b], g)
                pltpu.async_copy(rows[b], out_sh.at[dstg.at[g]], sems,
                                 add=True)
            for b in range(NB):
                pltpu.make_async_copy(rows[b], out_sh.at[dstg.at[g0 + b]],
                                      sems).wait()

            @pl.when(i < NGRP // NB - 1)
            def _():
                for b in range(NB):
                    pltpu.async_copy(feat.at[srcg.at[g0 + NB + b]], rows[b],
                                     sem)
            return carry
        with jax.named_scope("msg"):
            lax.fori_loop(0, NGRP // NB, mloop, 0)
        plsc.subcore_barrier()

        @pl.when(s < NZCH)
        def _():
            pltpu.sync_copy(out_sh.at[pl.ds(s * ZCH, ZCH)],
                            out_parts.at[c, pl.ds(s * ZCH, ZCH)])

    return pl.kernel(
        body,
        out_type=jax.ShapeDtypeStruct((NCORES, N, D), _f32),
        mesh=_MESH,
        scratch_types=[
            pltpu.VMEM((N,), _f32),
            pltpu.VMEM((N,), _f32),
            pltpu.VMEM((NGRP, G), _i32),
            pltpu.VMEM((NGRP, G), _i32),
            pltpu.VMEM((EW,), _f32),
            pltpu.VMEM((N,), _f32),
            pltpu.VMEM((G, D), _f32),
            pltpu.VMEM((G, D), _f32),
            pltpu.VMEM((G, D), _f32),
            pltpu.VMEM((G, D), _f32),
            pltpu.VMEM((G, D), _f32),
            pltpu.VMEM((16,), _f32),
            pltpu.VMEM_SHARED((N,), _f32),
            pltpu.VMEM_SHARED((N, D), _f32),
            pltpu.SemaphoreType.DMA,
            pltpu.SemaphoreType.DMA,
        ],
        compiler_params=pltpu.CompilerParams(
            needs_layout_passes=False, use_tc_tiling_on_sc=False),
    )


_sc_layer1 = _sc_edge_pass(D1, rows_scale_rowmode=True)
_sc_layer2 = _sc_edge_pass(D2, rows_scale_rowmode=False)


def kernel(x, edge_index, batch, W1, a1s, a1d, b1, W2, a2s, a2d, b2, fcw, fcb):
    ei4 = edge_index.astype(_i32).reshape(2, NTILES, NGRP, G)
    idc = jnp.arange(N, dtype=_i32).reshape(NGRP, G)

    zr1 = jnp.zeros((ZCH, D1), _f32)
    zr2 = jnp.zeros((ZCH, D2), _f32)

    h1, asad1, cvec1 = _tc_head(x, W1, a1s, a1d)
    p1 = _sc_layer1(ei4, asad1, cvec1, h1, idc, zr1)

    h2, asad2, cvec2 = _tc_mid(p1, b1.reshape(1, D1), W2, a2s, a2d)
    p2 = _sc_layer2(ei4, asad2, cvec2, h2, idc, zr2)

    out = _tc_end(p2, b2.reshape(1, D2),
                  batch.reshape(1, N).astype(_i32), fcw, fcb.reshape(1, 1))
    return out.reshape(NG)


# speedup vs baseline: 1.0575x; 1.0575x over previous
import functools

import jax
import jax.numpy as jnp
from jax import lax
from jax.experimental import pallas as pl
from jax.experimental.pallas import tpu as pltpu
from jax.experimental.pallas import tpu_sc as plsc

N = 10000
E = 320000
IN_CH = 128
D1 = 64
D2 = 8
NG = 16
NCORES = 2
NSUB = 16
NTILES = NCORES * NSUB
EW = E // NTILES
G = 80
NGRP = EW // G
ZCH = 1000
NZCH = N // ZCH
NB = 5

_f32 = jnp.float32
_i32 = jnp.int32


def _tc_head_body(x_ref, w_ref, avs_ref, avd_ref, h_ref, asad_ref, cm_ref):
    h = jnp.dot(x_ref[...], w_ref[...], preferred_element_type=_f32)
    h_ref[...] = h
    a_s = jnp.sum(h * avs_ref[...], axis=1)
    a_d = jnp.sum(h * avd_ref[...], axis=1)
    asad_ref[0:1, :] = a_s[None]
    asad_ref[1:2, :] = a_d[None]
    cm = jnp.max(a_s) + jnp.max(a_d)
    cm_ref[...] = jnp.zeros((1, 16), _f32) + jnp.maximum(cm, 0.2 * cm)


def _tc_head(x, W1, a1s, a1d):
    return pl.pallas_call(
        _tc_head_body,
        out_shape=[
            jax.ShapeDtypeStruct((N, D1), _f32),
            jax.ShapeDtypeStruct((2, N), _f32),
            jax.ShapeDtypeStruct((1, 16), _f32),
        ],
    )(x, W1, a1s, a1d)


def _tc_mid_body(p_ref, b1_ref, w2_ref, avs_ref, avd_ref,
                 h2_ref, asad_ref, cm_ref):
    o = jnp.maximum(p_ref[0] + p_ref[1] + b1_ref[...], 0.0)
    h2 = jnp.dot(o, w2_ref[...], preferred_element_type=_f32)
    h2_ref[...] = h2
    a_s = jnp.sum(h2 * avs_ref[...], axis=1)
    a_d = jnp.sum(h2 * avd_ref[...], axis=1)
    asad_ref[0:1, :] = a_s[None]
    asad_ref[1:2, :] = a_d[None]
    cm = jnp.max(a_s) + jnp.max(a_d)
    cm_ref[...] = jnp.zeros((1, 16), _f32) + jnp.maximum(cm, 0.2 * cm)


def _tc_mid(p, b1, W2, a2s, a2d):
    return pl.pallas_call(
        _tc_mid_body,
        out_shape=[
            jax.ShapeDtypeStruct((N, D2), _f32),
            jax.ShapeDtypeStruct((2, N), _f32),
            jax.ShapeDtypeStruct((1, 16), _f32),
        ],
    )(p, b1, W2, a2s, a2d)


def _tc_end_body(p_ref, b2_ref, batch_ref, fcw_ref, fcb_ref, out_ref):
    h = jnp.maximum(p_ref[0] + p_ref[1] + b2_ref[...], 0.0)
    gids = lax.broadcasted_iota(_i32, (NG, N), 0)
    oh = (gids == batch_ref[...]).astype(_f32)
    counts = jnp.sum(oh, axis=1, keepdims=True)
    sums = jnp.dot(oh, h, preferred_element_type=_f32)
    pooled = sums / jnp.maximum(counts, 1.0)
    logits = jnp.sum(pooled * fcw_ref[...], axis=1, keepdims=True) + fcb_ref[...]
    out_ref[...] = 2.0 / (1.0 + jnp.exp(-logits)) - 1.0


def _tc_end(p, b2, batch2, fcw, fcb):
    return pl.pallas_call(
        _tc_end_body,
        out_shape=jax.ShapeDtypeStruct((NG, 1), _f32),
    )(p, b2, batch2, fcw, fcb)


_MESH = plsc.VectorSubcoreMesh(core_axis_name="c", subcore_axis_name="s")


def _sc_edge_pass(D, rows_scale_rowmode, nb):
    def body(ei4, asad, cvec, feat, idc, zrows, out_parts, *scr):
        asv, adv, srcg, dstg, exv, denv = scr[:6]
        rows = scr[6:6 + nb]
        cvecv = scr[6 + nb]
        den_sh, out_sh, sem, sems = scr[7 + nb:]
        c = lax.axis_index("c")
        s = lax.axis_index("s")
        w = c * NSUB + s

        pltpu.sync_copy(asad.at[0], asv)
        pltpu.sync_copy(asad.at[1], adv)
        pltpu.sync_copy(cvec.at[0], cvecv)
        cv = cvecv[...]

        zero16 = jnp.zeros((16,), _f32)

        def zero_den(i, carry):
            denv[pl.ds(i * 16, 16)] = zero16
            return carry
        lax.fori_loop(0, N // 16, zero_den, 0)

        @pl.when(s == 0)
        def _():
            pltpu.sync_copy(denv, den_sh)

        @pl.when(s < NZCH)
        def _():
            pltpu.sync_copy(zrows, out_sh.at[pl.ds(s * ZCH, ZCH)])
        plsc.subcore_barrier()

        for slab in (s + NSUB * (1 - c), s + NSUB * c):
          with jax.named_scope("p1_slab"):
            pltpu.sync_copy(ei4.at[0, slab], srcg)
            pltpu.sync_copy(ei4.at[1, slab], dstg)

            def p1(g, carry):
                for k in range(G // 16):
                    si = srcg[g, pl.ds(k * 16, 16)]
                    di = dstg[g, pl.ds(k * 16, 16)]
                    av = plsc.load_gather(asv, [si])
                    bv = plsc.load_gather(adv, [di])
                    e = av + bv
                    e = jnp.maximum(e, 0.2 * e)
                    ex = jnp.exp(e - cv)
                    exv[pl.ds(g * G + k * 16, 16)] = ex
                    plsc.addupdate_scatter(denv, [di], ex)
                return carry
            lax.fori_loop(0, NGRP, p1, 0)

        pltpu.sync_copy(idc, srcg)

        def dadd(i, carry):
            for b in range(NB):
                pltpu.async_copy(denv.at[pl.ds((i * NB + b) * G, G)],
                                 den_sh.at[srcg.at[i * NB + b]], sems,
                                 add=True)

            @pl.when(i > 0)
            def _():
                for b in range(NB):
                    j = i * NB + b - NB
                    pltpu.make_async_copy(denv.at[pl.ds(j * G, G)],
                                          den_sh.at[srcg.at[j]], sems).wait()
            return carry
        with jax.named_scope("dmerge"):
            lax.fori_loop(0, NGRP // NB, dadd, 0)
        for b in range(NB):
            j = NGRP - NB + b
            pltpu.make_async_copy(denv.at[pl.ds(j * G, G)],
                                  den_sh.at[srcg.at[j]], sems).wait()
        plsc.subcore_barrier()

        pltpu.sync_copy(ei4.at[0, w], srcg)
        pltpu.sync_copy(ei4.at[1, w], dstg)
        pltpu.sync_copy(den_sh, denv)

        if rows_scale_rowmode:
            def scale(rbuf, g):
                for k in range(G // 16):
                    di = dstg[g, pl.ds(k * 16, 16)]
                    d = plsc.load_gather(denv, [di])
                    av = exv[pl.ds(g * G + k * 16, 16)] / (d + 1e-16)
                    for e16 in range(16):
                        a = av[e16]
                        e = k * 16 + e16
                        for j in range(D // 16):
                            rbuf[e, pl.ds(j * 16, 16)] = rbuf[e, pl.ds(j * 16, 16)] * a
        else:
            def scale(rbuf, g):
                for k in range(G // 16):
                    di = dstg[g, pl.ds(k * 16, 16)]
                    d = plsc.load_gather(denv, [di])
                    av = exv[pl.ds(g * G + k * 16, 16)] / (d + 1e-16)
                    ridx = k * 16 + lax.iota(_i32, 16)
                    for j in range(D):
                        cj = jnp.full((16,), j, _i32)
                        col = plsc.load_gather(rbuf, [ridx, cj])
                        plsc.store_scatter(rbuf, [ridx, cj], col * av)

        for b in range(nb):
            pltpu.async_copy(feat.at[srcg.at[b]], rows[b], sem)

        def mloop(i, carry):
            g0 = i * nb
            for b in range(nb):
                g = g0 + b
                pltpu.make_async_copy(feat.at[srcg.at[g]], rows[b], sem).wait()
                scale(rows[b], g)
                pltpu.async_copy(rows[b], out_sh.at[dstg.at[g]], sems,
                                 add=True)
            for b in range(nb):
                pltpu.make_async_copy(rows[b], out_sh.at[dstg.at[g0 + b]],
                                      sems).wait()

                @pl.when(i < NGRP // nb - 1)
                def _():
                    pltpu.async_copy(feat.at[srcg.at[g0 + nb + b]], rows[b],
                                     sem)
            return carry
        with jax.named_scope("msg"):
            lax.fori_loop(0, NGRP // nb, mloop, 0)
        plsc.subcore_barrier()

        @pl.when(s < NZCH)
        def _():
            pltpu.sync_copy(out_sh.at[pl.ds(s * ZCH, ZCH)],
                            out_parts.at[c, pl.ds(s * ZCH, ZCH)])

    return pl.kernel(
        body,
        out_type=jax.ShapeDtypeStruct((NCORES, N, D), _f32),
        mesh=_MESH,
        scratch_types=[
            pltpu.VMEM((N,), _f32),
            pltpu.VMEM((N,), _f32),
            pltpu.VMEM((NGRP, G), _i32),
            pltpu.VMEM((NGRP, G), _i32),
            pltpu.VMEM((EW,), _f32),
            pltpu.VMEM((N,), _f32),
            *[pltpu.VMEM((G, D), _f32) for _ in range(nb)],
            pltpu.VMEM((16,), _f32),
            pltpu.VMEM_SHARED((N,), _f32),
            pltpu.VMEM_SHARED((N, D), _f32),
            pltpu.SemaphoreType.DMA,
            pltpu.SemaphoreType.DMA,
        ],
        compiler_params=pltpu.CompilerParams(
            needs_layout_passes=False, use_tc_tiling_on_sc=False),
    )


_sc_layer1 = _sc_edge_pass(D1, rows_scale_rowmode=True, nb=5)
_sc_layer2 = _sc_edge_pass(D2, rows_scale_rowmode=False, nb=25)


def kernel(x, edge_index, batch, W1, a1s, a1d, b1, W2, a2s, a2d, b2, fcw, fcb):
    ei4 = edge_index.astype(_i32).reshape(2, NTILES, NGRP, G)
    idc = jnp.arange(N, dtype=_i32).reshape(NGRP, G)

    zr1 = jnp.zeros((ZCH, D1), _f32)
    zr2 = jnp.zeros((ZCH, D2), _f32)

    h1, asad1, cvec1 = _tc_head(x, W1, a1s, a1d)
    p1 = _sc_layer1(ei4, asad1, cvec1, h1, idc, zr1)

    h2, asad2, cvec2 = _tc_mid(p1, b1.reshape(1, D1), W2, a2s, a2d)
    p2 = _sc_layer2(ei4, asad2, cvec2, h2, idc, zr2)

    out = _tc_end(p2, b2.reshape(1, D2),
                  batch.reshape(1, N).astype(_i32), fcw, fcb.reshape(1, 1))
    return out.reshape(NG)


# speedup vs baseline: 1.0635x; 1.0057x over previous
import functools

import jax
import jax.numpy as jnp
from jax import lax
from jax.experimental import pallas as pl
from jax.experimental.pallas import tpu as pltpu
from jax.experimental.pallas import tpu_sc as plsc

N = 10000
E = 320000
IN_CH = 128
D1 = 64
D2 = 8
NG = 16
NCORES = 2
NSUB = 16
NTILES = NCORES * NSUB
EW = E // NTILES
G = 80
NGRP = EW // G
ZCH = 1000
NZCH = N // ZCH
NB = 5

_f32 = jnp.float32
_i32 = jnp.int32


def _tc_head_body(x_ref, w_ref, avs_ref, avd_ref, h_ref, asad_ref, cm_ref):
    h = jnp.dot(x_ref[...], w_ref[...], preferred_element_type=_f32)
    h_ref[...] = h
    a_s = jnp.sum(h * avs_ref[...], axis=1)
    a_d = jnp.sum(h * avd_ref[...], axis=1)
    asad_ref[0:1, :] = a_s[None]
    asad_ref[1:2, :] = a_d[None]
    cm = jnp.max(a_s) + jnp.max(a_d)
    cm_ref[...] = jnp.zeros((1, 16), _f32) + jnp.maximum(cm, 0.2 * cm)


def _tc_head(x, W1, a1s, a1d):
    return pl.pallas_call(
        _tc_head_body,
        out_shape=[
            jax.ShapeDtypeStruct((N, D1), _f32),
            jax.ShapeDtypeStruct((2, N), _f32),
            jax.ShapeDtypeStruct((1, 16), _f32),
        ],
    )(x, W1, a1s, a1d)


def _tc_mid_body(p_ref, b1_ref, w2_ref, avs_ref, avd_ref,
                 h2_ref, asad_ref, cm_ref):
    o = jnp.maximum(p_ref[0] + p_ref[1] + b1_ref[...], 0.0)
    h2 = jnp.dot(o, w2_ref[...], preferred_element_type=_f32)
    h2_ref[...] = h2
    a_s = jnp.sum(h2 * avs_ref[...], axis=1)
    a_d = jnp.sum(h2 * avd_ref[...], axis=1)
    asad_ref[0:1, :] = a_s[None]
    asad_ref[1:2, :] = a_d[None]
    cm = jnp.max(a_s) + jnp.max(a_d)
    cm_ref[...] = jnp.zeros((1, 16), _f32) + jnp.maximum(cm, 0.2 * cm)


def _tc_mid(p, b1, W2, a2s, a2d):
    return pl.pallas_call(
        _tc_mid_body,
        out_shape=[
            jax.ShapeDtypeStruct((N, D2), _f32),
            jax.ShapeDtypeStruct((2, N), _f32),
            jax.ShapeDtypeStruct((1, 16), _f32),
        ],
    )(p, b1, W2, a2s, a2d)


def _tc_end_body(p_ref, b2_ref, batch_ref, fcw_ref, fcb_ref, out_ref):
    h = jnp.maximum(p_ref[0] + p_ref[1] + b2_ref[...], 0.0)
    gids = lax.broadcasted_iota(_i32, (NG, N), 0)
    oh = (gids == batch_ref[...]).astype(_f32)
    counts = jnp.sum(oh, axis=1, keepdims=True)
    sums = jnp.dot(oh, h, preferred_element_type=_f32)
    pooled = sums / jnp.maximum(counts, 1.0)
    logits = jnp.sum(pooled * fcw_ref[...], axis=1, keepdims=True) + fcb_ref[...]
    out_ref[...] = 2.0 / (1.0 + jnp.exp(-logits)) - 1.0


def _tc_end(p, b2, batch2, fcw, fcb):
    return pl.pallas_call(
        _tc_end_body,
        out_shape=jax.ShapeDtypeStruct((NG, 1), _f32),
    )(p, b2, batch2, fcw, fcb)


_MESH = plsc.VectorSubcoreMesh(core_axis_name="c", subcore_axis_name="s")


def _sc_edge_pass(D, rows_scale_rowmode, nb):
    def body(ei4, asad, cvec, feat, idc, zrows, out_parts, *scr):
        asv, adv, srcg, dstg, exv, denv = scr[:6]
        rows = scr[6:6 + nb]
        cvecv = scr[6 + nb]
        den_sh, out_sh, sem, sems = scr[7 + nb:]
        c = lax.axis_index("c")
        s = lax.axis_index("s")
        w = c * NSUB + s

        pltpu.sync_copy(asad.at[0], asv)
        pltpu.sync_copy(asad.at[1], adv)
        pltpu.sync_copy(cvec.at[0], cvecv)
        cv = cvecv[...]

        zero16 = jnp.zeros((16,), _f32)

        def zero_den(i, carry):
            for u in range(5):
                denv[pl.ds((i * 5 + u) * 16, 16)] = zero16
            return carry
        lax.fori_loop(0, N // 80, zero_den, 0)

        @pl.when(s == 0)
        def _():
            pltpu.sync_copy(denv, den_sh)

        @pl.when(s < NZCH)
        def _():
            pltpu.sync_copy(zrows, out_sh.at[pl.ds(s * ZCH, ZCH)])
        plsc.subcore_barrier()

        for slab in (s + NSUB * (1 - c), s + NSUB * c):
          with jax.named_scope("p1_slab"):
            pltpu.sync_copy(ei4.at[0, slab], srcg)
            pltpu.sync_copy(ei4.at[1, slab], dstg)

            def p1(i, carry):
                for u in range(5):
                    g = i * 5 + u
                    for k in range(G // 16):
                        si = srcg[g, pl.ds(k * 16, 16)]
                        di = dstg[g, pl.ds(k * 16, 16)]
                        av = plsc.load_gather(asv, [si])
                        bv = plsc.load_gather(adv, [di])
                        e = av + bv
                        e = jnp.maximum(e, 0.2 * e)
                        ex = jnp.exp(e - cv)
                        exv[pl.ds(g * G + k * 16, 16)] = ex
                        plsc.addupdate_scatter(denv, [di], ex)
                return carry
            lax.fori_loop(0, NGRP // 5, p1, 0)

        pltpu.sync_copy(idc, srcg)

        def dadd(i, carry):
            for b in range(NB):
                pltpu.async_copy(denv.at[pl.ds((i * NB + b) * G, G)],
                                 den_sh.at[srcg.at[i * NB + b]], sems,
                                 add=True)

            @pl.when(i > 0)
            def _():
                for b in range(NB):
                    j = i * NB + b - NB
                    pltpu.make_async_copy(denv.at[pl.ds(j * G, G)],
                                          den_sh.at[srcg.at[j]], sems).wait()
            return carry
        with jax.named_scope("dmerge"):
            lax.fori_loop(0, NGRP // NB, dadd, 0)
        for b in range(NB):
            j = NGRP - NB + b
            pltpu.make_async_copy(denv.at[pl.ds(j * G, G)],
                                  den_sh.at[srcg.at[j]], sems).wait()
        plsc.subcore_barrier()

        pltpu.sync_copy(ei4.at[0, w], srcg)
        pltpu.sync_copy(ei4.at[1, w], dstg)
        pltpu.sync_copy(den_sh, denv)

        if rows_scale_rowmode:
            def scale(rbuf, g):
                for k in range(G // 16):
                    di = dstg[g, pl.ds(k * 16, 16)]
                    d = plsc.load_gather(denv, [di])
                    av = exv[pl.ds(g * G + k * 16, 16)] / (d + 1e-16)
                    for e16 in range(16):
                        a = av[e16]
                        e = k * 16 + e16
                        for j in range(D // 16):
                            rbuf[e, pl.ds(j * 16, 16)] = rbuf[e, pl.ds(j * 16, 16)] * a
        else:
            def scale(rbuf, g):
                for k in range(G // 16):
                    di = dstg[g, pl.ds(k * 16, 16)]
                    d = plsc.load_gather(denv, [di])
                    av = exv[pl.ds(g * G + k * 16, 16)] / (d + 1e-16)
                    ridx = k * 16 + lax.iota(_i32, 16)
                    for j in range(D):
                        cj = jnp.full((16,), j, _i32)
                        col = plsc.load_gather(rbuf, [ridx, cj])
                        plsc.store_scatter(rbuf, [ridx, cj], col * av)

        for b in range(nb):
            pltpu.async_copy(feat.at[srcg.at[b]], rows[b], sem)

        def mloop(i, carry):
            g0 = i * nb
            for b in range(nb):
                g = g0 + b
                pltpu.make_async_copy(feat.at[srcg.at[g]], rows[b], sem).wait()
                scale(rows[b], g)
                pltpu.async_copy(rows[b], out_sh.at[dstg.at[g]], sems,
                                 add=True)
            for b in range(nb):
                pltpu.make_async_copy(rows[b], out_sh.at[dstg.at[g0 + b]],
                                      sems).wait()

                @pl.when(i < NGRP // nb - 1)
                def _():
                    pltpu.async_copy(feat.at[srcg.at[g0 + nb + b]], rows[b],
                                     sem)
            return carry
        with jax.named_scope("msg"):
            lax.fori_loop(0, NGRP // nb, mloop, 0)
        plsc.subcore_barrier()

        @pl.when(s < NZCH)
        def _():
            pltpu.sync_copy(out_sh.at[pl.ds(s * ZCH, ZCH)],
                            out_parts.at[c, pl.ds(s * ZCH, ZCH)])

    return pl.kernel(
        body,
        out_type=jax.ShapeDtypeStruct((NCORES, N, D), _f32),
        mesh=_MESH,
        scratch_types=[
            pltpu.VMEM((N,), _f32),
            pltpu.VMEM((N,), _f32),
            pltpu.VMEM((NGRP, G), _i32),
            pltpu.VMEM((NGRP, G), _i32),
            pltpu.VMEM((EW,), _f32),
            pltpu.VMEM((N,), _f32),
            *[pltpu.VMEM((G, D), _f32) for _ in range(nb)],
            pltpu.VMEM((16,), _f32),
            pltpu.VMEM_SHARED((N,), _f32),
            pltpu.VMEM_SHARED((N, D), _f32),
            pltpu.SemaphoreType.DMA,
            pltpu.SemaphoreType.DMA,
        ],
        compiler_params=pltpu.CompilerParams(
            needs_layout_passes=False, use_tc_tiling_on_sc=False),
    )


_sc_layer1 = _sc_edge_pass(D1, rows_scale_rowmode=True, nb=5)
_sc_layer2 = _sc_edge_pass(D2, rows_scale_rowmode=False, nb=25)


def kernel(x, edge_index, batch, W1, a1s, a1d, b1, W2, a2s, a2d, b2, fcw, fcb):
    ei4 = edge_index.astype(_i32).reshape(2, NTILES, NGRP, G)
    idc = jnp.arange(N, dtype=_i32).reshape(NGRP, G)

    zr1 = jnp.zeros((ZCH, D1), _f32)
    zr2 = jnp.zeros((ZCH, D2), _f32)

    h1, asad1, cvec1 = _tc_head(x, W1, a1s, a1d)
    p1 = _sc_layer1(ei4, asad1, cvec1, h1, idc, zr1)

    h2, asad2, cvec2 = _tc_mid(p1, b1.reshape(1, D1), W2, a2s, a2d)
    p2 = _sc_layer2(ei4, asad2, cvec2, h2, idc, zr2)

    out = _tc_end(p2, b2.reshape(1, D2),
                  batch.reshape(1, N).astype(_i32), fcw, fcb.reshape(1, 1))
    return out.reshape(NG)
